# baseline (speedup 1.0000x reference)
import dataclasses
import functools
import jax
import jax.numpy as jnp
from jax import lax
from jax.experimental import pallas as pl
from jax.experimental.pallas import tpu as pltpu
from jax.experimental.pallas import tpu_sc as plsc

_B = 1024
_G = 512
_A = 4
_OUT = 16

_NC = 2
_NS = 16
_NW = _NC * _NS
_GW = _G // _NW
_NBCH = 128
_NT = _NBCH // 16
_OPAD = _GW * _OUT + 1


def _body(xt_hbm, p_hbm, r15_hbm, out_hbm, xbuf0, xbuf1, pbuf, r15buf,
          obuf0, obuf1, xsem0, xsem1, osem0, osem1):
    xbufs = (xbuf0, xbuf1)
    obufs = (obuf0, obuf1)
    xsems = (xsem0, xsem1)
    osems = (osem0, osem1)
    wid = lax.axis_index("s") * _NC + lax.axis_index("c")
    g0 = wid * _GW

    pltpu.sync_copy(p_hbm.at[pl.ds(g0 * (2 ** _A * _OUT), _GW * 2 ** _A * _OUT)],
                    pbuf)
    pltpu.sync_copy(r15_hbm.at[pl.ds(g0 * _OUT, _GW * _OUT)], r15buf)

    iota = lax.iota(jnp.int32, 16)
    fifteen = jnp.full((16,), 15, jnp.int32)

    def do_group(gi, t, xbuf, obuf):
        boff = t * 16
        v0 = xbuf[gi, 0, pl.ds(boff, 16)]
        v1 = xbuf[gi, 1, pl.ds(boff, 16)]
        v2 = xbuf[gi, 2, pl.ds(boff, 16)]
        v3 = xbuf[gi, 3, pl.ds(boff, 16)]
        p0 = jnp.full((16,), 1, jnp.int32)
        p1 = jnp.full((16,), 2, jnp.int32)
        p2 = jnp.full((16,), 4, jnp.int32)
        p3 = jnp.full((16,), 8, jnp.int32)

        v = [v0, v1, v2, v3]
        p = [p0, p1, p2, p3]
        for (i, j) in ((0, 1), (2, 3), (0, 2), (1, 3), (1, 2)):
            gt = v[i] > v[j]
            lo = jnp.where(gt, v[j], v[i])
            hi = jnp.where(gt, v[i], v[j])
            plo = jnp.where(gt, p[j], p[i])
            phi = jnp.where(gt, p[i], p[j])
            v[i], v[j] = lo, hi
            p[i], p[j] = plo, phi

        c0 = v[0]
        c1 = v[1] - v[0]
        c2 = v[2] - v[1]
        c3 = v[3] - v[2]
        i0 = fifteen
        i1 = fifteen - p[0]
        i2 = i1 - p[1]
        i3 = p[3]

        gbase = jnp.broadcast_to(gi * (2 ** _A * _OUT), (16,))
        b0 = gbase + i0
        b1 = gbase + i1
        b2 = gbase + i2
        b3 = gbase + i3
        rows = iota + boff
        colbase = gi * _OUT
        rv = r15buf[pl.ds(gi * _OUT, 16)]
        for l in range(_OUT):
            loff = l * 2 ** _A
            r0s = rv[l]
            r1 = plsc.load_gather(pbuf, [b1 + loff])
            r2 = plsc.load_gather(pbuf, [b2 + loff])
            r3 = plsc.load_gather(pbuf, [b3 + loff])
            acc = (c0 * r0s + c1 * r1) + (c2 * r2 + c3 * r3)
            cols = jnp.broadcast_to(colbase + l, (16,))
            plsc.store_scatter(obuf, [rows, cols], acc)

    nch = _B // _NBCH
    xh = [None, None]
    oh = [None, None]

    def start_x(bc):
        i = bc % 2
        xh[i] = pltpu.async_copy(
            xt_hbm.at[pl.ds(g0, _GW), :, pl.ds(bc * _NBCH, _NBCH)],
            xbufs[i], xsems[i])

    start_x(0)
    for bc in range(nch):
        i = bc % 2
        xh[i].wait()
        if bc + 1 < nch:
            start_x(bc + 1)
        if oh[i] is not None:
            oh[i].wait()

        @plsc.parallel_loop(0, _NT * _GW, step=1, unroll=4)
        def _(u):
            do_group(u & (_GW - 1), u >> 4, xbufs[i], obufs[i])

        oh[i] = pltpu.async_copy(
            obufs[i].at[:, pl.ds(0, _GW * _OUT)],
            out_hbm.at[pl.ds(bc * _NBCH, _NBCH), pl.ds(g0 * _OUT, _GW * _OUT)],
            osems[i])
    for h in oh:
        if h is not None:
            h.wait()


@jax.jit
def _run(Xt, params, r15):
    mesh = plsc.VectorSubcoreMesh(core_axis_name="c", subcore_axis_name="s")
    cp = pltpu.CompilerParams()
    if "needs_layout_passes" in pltpu.CompilerParams.__dataclass_fields__:
        cp = dataclasses.replace(cp, needs_layout_passes=False)
    f = functools.partial(
        pl.kernel,
        out_type=jax.ShapeDtypeStruct((_B, _G * _OUT), jnp.float32),
        mesh=mesh,
        compiler_params=cp,
        scratch_types=[
            pltpu.VMEM((_GW, _A, _NBCH), jnp.float32),
            pltpu.VMEM((_GW, _A, _NBCH), jnp.float32),
            pltpu.VMEM((_GW * 2 ** _A * _OUT,), jnp.float32),
            pltpu.VMEM((_GW * _OUT,), jnp.float32),
            pltpu.VMEM((_NBCH, _OPAD), jnp.float32),
            pltpu.VMEM((_NBCH, _OPAD), jnp.float32),
            pltpu.SemaphoreType.DMA,
            pltpu.SemaphoreType.DMA,
            pltpu.SemaphoreType.DMA,
            pltpu.SemaphoreType.DMA,
        ],
    )(_body)
    return f(Xt, params, r15)


def kernel(X, params):
    Xt = jnp.transpose(X.reshape(_B, _G, _A), (1, 2, 0))
    pt = jnp.transpose(params, (0, 2, 1)).reshape(-1)
    r15 = params[:, 15, :].reshape(-1)
    return _run(Xt, pt, r15)


# speedup vs baseline: 1.0571x; 1.0571x over previous
import dataclasses
import functools
import jax
import jax.numpy as jnp
from jax import lax
from jax.experimental import pallas as pl
from jax.experimental.pallas import tpu as pltpu
from jax.experimental.pallas import tpu_sc as plsc

_B = 1024
_G = 512
_A = 4
_OUT = 16

_NC = 2
_NS = 16
_NW = _NC * _NS
_GW = _G // _NW
_NBCH = 128
_NT = _NBCH // 16
_OPAD = _GW * _OUT + 1


def _body(xt_hbm, p_hbm, out_hbm, xbuf0, xbuf1, pbuf,
          obuf0, obuf1, xsem0, xsem1, osem0, osem1):
    xbufs = (xbuf0, xbuf1)
    obufs = (obuf0, obuf1)
    xsems = (xsem0, xsem1)
    osems = (osem0, osem1)
    wid = lax.axis_index("s") * _NC + lax.axis_index("c")
    g0 = wid * _GW

    pltpu.sync_copy(p_hbm.at[pl.ds(g0 * (2 ** _A * _OUT), _GW * 2 ** _A * _OUT)],
                    pbuf)

    iota = lax.iota(jnp.int32, 16)
    fifteen = jnp.full((16,), 15, jnp.int32)

    def do_group(gi, t, xbuf, obuf):
        boff = t * 16
        v0 = xbuf[gi, 0, pl.ds(boff, 16)]
        v1 = xbuf[gi, 1, pl.ds(boff, 16)]
        v2 = xbuf[gi, 2, pl.ds(boff, 16)]
        v3 = xbuf[gi, 3, pl.ds(boff, 16)]
        p0 = jnp.full((16,), 1, jnp.int32)
        p1 = jnp.full((16,), 2, jnp.int32)
        p2 = jnp.full((16,), 4, jnp.int32)
        p3 = jnp.full((16,), 8, jnp.int32)

        v = [v0, v1, v2, v3]
        p = [p0, p1, p2, p3]
        for (i, j) in ((0, 1), (2, 3), (0, 2), (1, 3), (1, 2)):
            gt = v[i] > v[j]
            lo = jnp.where(gt, v[j], v[i])
            hi = jnp.where(gt, v[i], v[j])
            plo = jnp.where(gt, p[j], p[i])
            phi = jnp.where(gt, p[i], p[j])
            v[i], v[j] = lo, hi
            p[i], p[j] = plo, phi

        c0 = v[0]
        c1 = v[1] - v[0]
        c2 = v[2] - v[1]
        c3 = v[3] - v[2]
        i1 = fifteen - p[0]
        i2 = i1 - p[1]
        i3 = p[3]

        rows = iota + boff
        colbase = gi * _OUT
        pbase = gi * (2 ** _A * _OUT)
        for l in range(_OUT):
            sl = pbuf.at[pl.ds(pbase + l * 2 ** _A, 2 ** _A)]
            r0s = sl[...][15]
            r1 = plsc.load_gather(sl, [i1])
            r2 = plsc.load_gather(sl, [i2])
            r3 = plsc.load_gather(sl, [i3])
            acc = (c0 * r0s + c1 * r1) + (c2 * r2 + c3 * r3)
            cols = jnp.broadcast_to(colbase + l, (16,))
            plsc.store_scatter(obuf, [rows, cols], acc)

    nch = _B // _NBCH
    xh = [None, None]
    oh = [None, None]

    def start_x(bc):
        i = bc % 2
        xh[i] = pltpu.async_copy(
            xt_hbm.at[pl.ds(g0, _GW), :, pl.ds(bc * _NBCH, _NBCH)],
            xbufs[i], xsems[i])

    start_x(0)
    for bc in range(nch):
        i = bc % 2
        xh[i].wait()
        if bc + 1 < nch:
            start_x(bc + 1)
        if oh[i] is not None:
            oh[i].wait()

        @plsc.parallel_loop(0, _NT * _GW, step=1, unroll=4)
        def _(u):
            do_group(u & (_GW - 1), u >> 4, xbufs[i], obufs[i])

        oh[i] = pltpu.async_copy(
            obufs[i].at[:, pl.ds(0, _GW * _OUT)],
            out_hbm.at[pl.ds(bc * _NBCH, _NBCH), pl.ds(g0 * _OUT, _GW * _OUT)],
            osems[i])
    for h in oh:
        if h is not None:
            h.wait()


@jax.jit
def _run(Xt, params):
    mesh = plsc.VectorSubcoreMesh(core_axis_name="c", subcore_axis_name="s")
    cp = pltpu.CompilerParams()
    if "needs_layout_passes" in pltpu.CompilerParams.__dataclass_fields__:
        cp = dataclasses.replace(cp, needs_layout_passes=False)
    f = functools.partial(
        pl.kernel,
        out_type=jax.ShapeDtypeStruct((_B, _G * _OUT), jnp.float32),
        mesh=mesh,
        compiler_params=cp,
        scratch_types=[
            pltpu.VMEM((_GW, _A, _NBCH), jnp.float32),
            pltpu.VMEM((_GW, _A, _NBCH), jnp.float32),
            pltpu.VMEM((_GW * 2 ** _A * _OUT,), jnp.float32),
            pltpu.VMEM((_NBCH, _OPAD), jnp.float32),
            pltpu.VMEM((_NBCH, _OPAD), jnp.float32),
            pltpu.SemaphoreType.DMA,
            pltpu.SemaphoreType.DMA,
            pltpu.SemaphoreType.DMA,
            pltpu.SemaphoreType.DMA,
        ],
    )(_body)
    return f(Xt, params)


def kernel(X, params):
    Xt = jnp.transpose(X.reshape(_B, _G, _A), (1, 2, 0))
    pt = jnp.transpose(params, (0, 2, 1)).reshape(-1)
    return _run(Xt, pt)


# speedup vs baseline: 1.9122x; 1.8089x over previous
import dataclasses
import functools
import jax
import jax.numpy as jnp
from jax import lax
from jax.experimental import pallas as pl
from jax.experimental.pallas import tpu as pltpu
from jax.experimental.pallas import tpu_sc as plsc

_B = 1024
_G = 512
_A = 4
_OUT = 16

_NC = 2
_NS = 16
_NW = _NC * _NS
_GW = _G // _NW
_NBCH = 128
_NT = _NBCH // 16


def _body(xt_hbm, p_hbm, out_hbm, xbuf0, xbuf1, pbuf,
          obuf0, obuf1, xsem0, xsem1, osem0, osem1):
    xbufs = (xbuf0, xbuf1)
    obufs = (obuf0, obuf1)
    xsems = (xsem0, xsem1)
    osems = (osem0, osem1)
    wid = lax.axis_index("s") * _NC + lax.axis_index("c")
    g0 = wid * _GW

    pltpu.sync_copy(p_hbm.at[pl.ds(g0 * (2 ** _A * _OUT), _GW * 2 ** _A * _OUT)],
                    pbuf)

    iota = lax.iota(jnp.int32, 16)
    fifteen = jnp.full((16,), 15, jnp.int32)

    def do_group(gi, t, xbuf, obuf):
        boff = t * 16
        v0 = xbuf[gi, 0, pl.ds(boff, 16)]
        v1 = xbuf[gi, 1, pl.ds(boff, 16)]
        v2 = xbuf[gi, 2, pl.ds(boff, 16)]
        v3 = xbuf[gi, 3, pl.ds(boff, 16)]
        p0 = jnp.full((16,), 1, jnp.int32)
        p1 = jnp.full((16,), 2, jnp.int32)
        p2 = jnp.full((16,), 4, jnp.int32)
        p3 = jnp.full((16,), 8, jnp.int32)

        v = [v0, v1, v2, v3]
        p = [p0, p1, p2, p3]
        for (i, j) in ((0, 1), (2, 3), (0, 2), (1, 3), (1, 2)):
            gt = v[i] > v[j]
            lo = jnp.where(gt, v[j], v[i])
            hi = jnp.where(gt, v[i], v[j])
            plo = jnp.where(gt, p[j], p[i])
            phi = jnp.where(gt, p[i], p[j])
            v[i], v[j] = lo, hi
            p[i], p[j] = plo, phi

        c0 = v[0]
        c1 = v[1] - v[0]
        c2 = v[2] - v[1]
        c3 = v[3] - v[2]
        i1 = fifteen - p[0]
        i2 = i1 - p[1]
        i3 = p[3]

        pbase = gi * (2 ** _A * _OUT)
        for l in range(_OUT):
            sl = pbuf.at[pl.ds(pbase + l * 2 ** _A, 2 ** _A)]
            r0s = sl[...][15]
            r1 = plsc.load_gather(sl, [i1])
            r2 = plsc.load_gather(sl, [i2])
            r3 = plsc.load_gather(sl, [i3])
            acc = (c0 * r0s + c1 * r1) + (c2 * r2 + c3 * r3)
            obuf[gi * _OUT + l, pl.ds(boff, 16)] = acc

    nch = _B // _NBCH
    xh = [None, None]
    oh = [None, None]

    def start_x(bc):
        i = bc % 2
        xh[i] = pltpu.async_copy(
            xt_hbm.at[pl.ds(g0, _GW), :, pl.ds(bc * _NBCH, _NBCH)],
            xbufs[i], xsems[i])

    start_x(0)
    for bc in range(nch):
        i = bc % 2
        xh[i].wait()
        if bc + 1 < nch:
            start_x(bc + 1)
        if oh[i] is not None:
            oh[i].wait()

        @plsc.parallel_loop(0, _NT * _GW, step=1, unroll=4)
        def _(u):
            do_group(u & (_GW - 1), u >> 4, xbufs[i], obufs[i])

        oh[i] = pltpu.async_copy(
            obufs[i],
            out_hbm.at[pl.ds(g0 * _OUT, _GW * _OUT), pl.ds(bc * _NBCH, _NBCH)],
            osems[i])
    for h in oh:
        if h is not None:
            h.wait()


@jax.jit
def _run(Xt, params):
    mesh = plsc.VectorSubcoreMesh(core_axis_name="c", subcore_axis_name="s")
    cp = pltpu.CompilerParams()
    if "needs_layout_passes" in pltpu.CompilerParams.__dataclass_fields__:
        cp = dataclasses.replace(cp, needs_layout_passes=False)
    f = functools.partial(
        pl.kernel,
        out_type=jax.ShapeDtypeStruct((_G * _OUT, _B), jnp.float32),
        mesh=mesh,
        compiler_params=cp,
        scratch_types=[
            pltpu.VMEM((_GW, _A, _NBCH), jnp.float32),
            pltpu.VMEM((_GW, _A, _NBCH), jnp.float32),
            pltpu.VMEM((_GW * 2 ** _A * _OUT,), jnp.float32),
            pltpu.VMEM((_GW * _OUT, _NBCH), jnp.float32),
            pltpu.VMEM((_GW * _OUT, _NBCH), jnp.float32),
            pltpu.SemaphoreType.DMA,
            pltpu.SemaphoreType.DMA,
            pltpu.SemaphoreType.DMA,
            pltpu.SemaphoreType.DMA,
        ],
    )(_body)
    return f(Xt, params)


def kernel(X, params):
    Xt = jnp.transpose(X.reshape(_B, _G, _A), (1, 2, 0))
    pt = jnp.transpose(params, (0, 2, 1)).reshape(-1)
    return _run(Xt, pt).T
